# baseline (speedup 1.0000x reference)
import functools

import jax
import jax.numpy as jnp
from jax import lax
from jax.experimental import pallas as pl
from jax.experimental.pallas import tpu as pltpu
from jax.experimental.pallas import tpu_sc as plsc

N = 10000
E = 320000
D = 128

NC = 2
NS = 16
NW = NC * NS

CH = 128
NCH = 80
NCHB = 16
NBLK = NCH // NCHB
EPT = NCH * CH
E_PAD = NW * EPT
DUMMY = N

AGG_ROWS = 10240
ZROWS = 32
ROWS_PER_TILE_ZERO = AGG_ROWS // NS
OCH = 80
NOCH = N // OCH

_MESH = plsc.VectorSubcoreMesh(
    core_axis_name="c", subcore_axis_name="s", num_cores=NC, num_subcores=NS
)


def _zero_fill(ref, nrows, ncols):
    zv = jnp.zeros((16,), jnp.float32)

    def body(i, _):
        r = i // (ncols // 16)
        c = (i % (ncols // 16)) * 16
        ref[r, pl.ds(c, 16)] = zv
        return 0

    lax.fori_loop(0, nrows * (ncols // 16), body, 0)


@functools.partial(
    pl.kernel,
    out_type=jax.ShapeDtypeStruct((NC, N, D), jnp.float32),
    mesh=_MESH,
    scratch_types=[
        pltpu.VMEM((NCH, CH), jnp.int32),
        pltpu.VMEM((CH, D), jnp.float32),
        pltpu.VMEM((ZROWS, D), jnp.float32),
        pltpu.VMEM_SHARED((AGG_ROWS, D), jnp.float32),
    ],
)
def _sc_deg(dst_hbm, out_hbm, dstv, onesv, zbuf, dega):
    cid = lax.axis_index("c")
    sid = lax.axis_index("s")
    wid = sid * NC + cid

    _zero_fill(zbuf, ZROWS, D)
    _zero_fill(onesv, CH, D)
    e0 = jnp.where(lax.iota(jnp.int32, 16) == 0, 1.0, 0.0).astype(jnp.float32)

    def fill_ones(i, _):
        onesv[i, pl.ds(0, 16)] = e0
        return 0

    lax.fori_loop(0, CH, fill_ones, 0)

    zbase = sid * ROWS_PER_TILE_ZERO

    def zero_chunk(t, _):
        pltpu.sync_copy(zbuf, dega.at[pl.ds(zbase + t * ZROWS, ZROWS)])
        return 0

    lax.fori_loop(0, ROWS_PER_TILE_ZERO // ZROWS, zero_chunk, 0)

    pltpu.sync_copy(dst_hbm.at[wid], dstv)
    plsc.subcore_barrier()

    def edge_chunk(j, _):
        pltpu.sync_copy(onesv, dega.at[dstv.at[j]], add=True)
        return 0

    lax.fori_loop(0, NCH, edge_chunk, 0)
    plsc.subcore_barrier()

    def out_chunk(t, _):
        j = t * NS + sid

        @pl.when(j < NOCH)
        def _():
            pltpu.sync_copy(
                dega.at[pl.ds(j * OCH, OCH)],
                out_hbm.at[cid, pl.ds(j * OCH, OCH)],
            )

        return 0

    lax.fori_loop(0, (NOCH + NS - 1) // NS, out_chunk, 0)


@functools.partial(
    pl.kernel,
    out_type=jax.ShapeDtypeStruct((NC, N, D), jnp.float32),
    mesh=_MESH,
    scratch_types=[
        pltpu.VMEM((NCH, CH), jnp.int32),
        pltpu.VMEM((NCH, CH), jnp.int32),
        pltpu.VMEM((CH, D), jnp.float32),
        pltpu.VMEM((ZROWS, D), jnp.float32),
        pltpu.VMEM_SHARED((AGG_ROWS, D), jnp.float32),
        pltpu.SemaphoreType.DMA,
    ],
)
def _sc_agg(y_hbm, src_hbm, dst_hbm, out_hbm, srcv, dstv, rows, zbuf, agg, sem):
    cid = lax.axis_index("c")
    sid = lax.axis_index("s")
    wid = sid * NC + cid

    _zero_fill(zbuf, ZROWS, D)

    zbase = sid * ROWS_PER_TILE_ZERO

    def zero_chunk(t, _):
        pltpu.sync_copy(zbuf, agg.at[pl.ds(zbase + t * ZROWS, ZROWS)])
        return 0

    lax.fori_loop(0, ROWS_PER_TILE_ZERO // ZROWS, zero_chunk, 0)

    pltpu.sync_copy(src_hbm.at[wid], srcv)
    pltpu.sync_copy(dst_hbm.at[wid], dstv)
    plsc.subcore_barrier()

    def edge_chunk(j, _):
        pltpu.async_copy(y_hbm.at[srcv.at[j]], rows, sem).wait()
        pltpu.sync_copy(rows, agg.at[dstv.at[j]], add=True)
        return 0

    lax.fori_loop(0, NCH, edge_chunk, 0)
    plsc.subcore_barrier()

    def out_chunk(t, _):
        j = t * NS + sid

        @pl.when(j < NOCH)
        def _():
            pltpu.sync_copy(
                agg.at[pl.ds(j * OCH, OCH)],
                out_hbm.at[cid, pl.ds(j * OCH, OCH)],
            )

        return 0

    lax.fori_loop(0, (NOCH + NS - 1) // NS, out_chunk, 0)


_RB = 1000
_GRID = N // _RB


def _dis_from_degp(degp_ref):
    deg = degp_ref[0, :, 0] + degp_ref[1, :, 0] + 1.0
    return lax.rsqrt(deg)[:, None]


def _tc1_body(degp_ref, x_ref, w_ref, y_ref):
    dis = _dis_from_degp(degp_ref)
    y_ref[...] = dis * jnp.dot(
        x_ref[...], w_ref[...], preferred_element_type=jnp.float32
    )


def _tc2_body(degp_ref, p_ref, y1_ref, w_ref, b_ref, y2_ref):
    dis = _dis_from_degp(degp_ref)
    aggr = p_ref[0] + p_ref[1] + y1_ref[...]
    h1 = jnp.maximum(dis * aggr + b_ref[...], 0.0)
    y2_ref[...] = dis * jnp.dot(h1, w_ref[...], preferred_element_type=jnp.float32)


def _tc3_body(degp_ref, p_ref, y2_ref, b_ref, h2_ref):
    dis = _dis_from_degp(degp_ref)
    aggr = p_ref[0] + p_ref[1] + y2_ref[...]
    h2_ref[...] = jnp.maximum(dis * aggr + b_ref[...], 0.0)


_degp_spec = pl.BlockSpec((NC, _RB, D), lambda i: (0, i, 0))
_row_spec = pl.BlockSpec((_RB, D), lambda i: (i, 0))
_part_spec = pl.BlockSpec((NC, _RB, D), lambda i: (0, i, 0))
_w_spec = pl.BlockSpec((D, D), lambda i: (0, 0))
_b_spec = pl.BlockSpec((D,), lambda i: (0,))

_tc1 = pl.pallas_call(
    _tc1_body,
    grid=(_GRID,),
    in_specs=[_degp_spec, _row_spec, _w_spec],
    out_specs=_row_spec,
    out_shape=jax.ShapeDtypeStruct((N, D), jnp.float32),
)

_tc2 = pl.pallas_call(
    _tc2_body,
    grid=(_GRID,),
    in_specs=[_degp_spec, _part_spec, _row_spec, _w_spec, _b_spec],
    out_specs=_row_spec,
    out_shape=jax.ShapeDtypeStruct((N, D), jnp.float32),
)

_tc3 = pl.pallas_call(
    _tc3_body,
    grid=(_GRID,),
    in_specs=[_degp_spec, _part_spec, _row_spec, _b_spec],
    out_specs=_row_spec,
    out_shape=jax.ShapeDtypeStruct((N, D), jnp.float32),
)


def kernel(x, edge_index, W1, b1, W2, b2):
    src = edge_index[0].astype(jnp.int32)
    dst = edge_index[1].astype(jnp.int32)
    pad = E_PAD - E
    src_p = jnp.concatenate([src, jnp.zeros((pad,), jnp.int32)]).reshape(NW, NCH, CH)
    dst_p = jnp.concatenate([dst, jnp.full((pad,), DUMMY, jnp.int32)]).reshape(
        NW, NCH, CH
    )

    degp = _sc_deg(dst_p)
    y1 = _tc1(degp, x, W1)
    p1 = _sc_agg(y1, src_p, dst_p)
    y2 = _tc2(degp, p1, y1, W2, b1)
    p2 = _sc_agg(y2, src_p, dst_p)
    h2 = _tc3(degp, p2, y2, b2)
    return h2


# speedup vs baseline: 1.1749x; 1.1749x over previous
import functools

import jax
import jax.numpy as jnp
from jax import lax
from jax.experimental import pallas as pl
from jax.experimental.pallas import tpu as pltpu
from jax.experimental.pallas import tpu_sc as plsc

N = 10000
E = 320000
D = 128

NC = 2
NS = 16
NW = NC * NS

CH = 128
NCH = 80
NCHB = 16
NBLK = NCH // NCHB
EPT = NCH * CH
E_PAD = NW * EPT
DUMMY = N

AGG_ROWS = 10240
ZROWS = 32
ROWS_PER_TILE_ZERO = AGG_ROWS // NS
OCH = 80
NOCH = N // OCH

_MESH = plsc.VectorSubcoreMesh(
    core_axis_name="c", subcore_axis_name="s", num_cores=NC, num_subcores=NS
)


def _zero_fill(ref, nrows, ncols):
    zv = jnp.zeros((16,), jnp.float32)

    def body(i, _):
        r = i // (ncols // 16)
        c = (i % (ncols // 16)) * 16
        ref[r, pl.ds(c, 16)] = zv
        return 0

    lax.fori_loop(0, nrows * (ncols // 16), body, 0)


@functools.partial(
    pl.kernel,
    out_type=jax.ShapeDtypeStruct((NC, N, D), jnp.float32),
    mesh=_MESH,
    scratch_types=[
        pltpu.VMEM((NCH, CH), jnp.int32),
        pltpu.VMEM((CH, D), jnp.float32),
        pltpu.VMEM((ZROWS, D), jnp.float32),
        pltpu.VMEM_SHARED((AGG_ROWS, D), jnp.float32),
    ],
)
def _sc_deg(dst_hbm, out_hbm, dstv, onesv, zbuf, dega):
    cid = lax.axis_index("c")
    sid = lax.axis_index("s")
    wid = sid * NC + cid

    _zero_fill(zbuf, ZROWS, D)
    _zero_fill(onesv, CH, D)
    e0 = jnp.where(lax.iota(jnp.int32, 16) == 0, 1.0, 0.0).astype(jnp.float32)

    def fill_ones(i, _):
        onesv[i, pl.ds(0, 16)] = e0
        return 0

    lax.fori_loop(0, CH, fill_ones, 0)

    zbase = sid * ROWS_PER_TILE_ZERO

    def zero_chunk(t, _):
        pltpu.sync_copy(zbuf, dega.at[pl.ds(zbase + t * ZROWS, ZROWS)])
        return 0

    lax.fori_loop(0, ROWS_PER_TILE_ZERO // ZROWS, zero_chunk, 0)

    pltpu.sync_copy(dst_hbm.at[wid], dstv)
    plsc.subcore_barrier()

    def edge_chunk(j, _):
        pltpu.sync_copy(onesv, dega.at[dstv.at[j]], add=True)
        return 0

    lax.fori_loop(0, NCH, edge_chunk, 0)
    plsc.subcore_barrier()

    def out_chunk(t, _):
        j = t * NS + sid

        @pl.when(j < NOCH)
        def _():
            pltpu.sync_copy(
                dega.at[pl.ds(j * OCH, OCH)],
                out_hbm.at[cid, pl.ds(j * OCH, OCH)],
            )

        return 0

    lax.fori_loop(0, (NOCH + NS - 1) // NS, out_chunk, 0)


@functools.partial(
    pl.kernel,
    out_type=jax.ShapeDtypeStruct((NC, N, D), jnp.float32),
    mesh=_MESH,
    scratch_types=[
        pltpu.VMEM((NCH, CH), jnp.int32),
        pltpu.VMEM((NCH, CH), jnp.int32),
        pltpu.VMEM((CH, D), jnp.float32),
        pltpu.VMEM((ZROWS, D), jnp.float32),
        pltpu.VMEM_SHARED((AGG_ROWS, D), jnp.float32),
        pltpu.SemaphoreType.DMA,
    ],
)
def _sc_agg(y_hbm, src_hbm, dst_hbm, out_hbm, srcv, dstv, rows, zbuf, agg, sem):
    cid = lax.axis_index("c")
    sid = lax.axis_index("s")
    wid = sid * NC + cid

    _zero_fill(zbuf, ZROWS, D)

    zbase = sid * ROWS_PER_TILE_ZERO

    def zero_chunk(t, _):
        pltpu.sync_copy(zbuf, agg.at[pl.ds(zbase + t * ZROWS, ZROWS)])
        return 0

    lax.fori_loop(0, ROWS_PER_TILE_ZERO // ZROWS, zero_chunk, 0)

    pltpu.sync_copy(src_hbm.at[wid], srcv)
    pltpu.sync_copy(dst_hbm.at[wid], dstv)
    plsc.subcore_barrier()

    def edge_chunk(j, _):
        pltpu.async_copy(y_hbm.at[srcv.at[j]], rows, sem).wait()
        pltpu.sync_copy(rows, agg.at[dstv.at[j]], add=True)
        return 0

    lax.fori_loop(0, NCH, edge_chunk, 0)
    plsc.subcore_barrier()

    def out_chunk(t, _):
        j = t * NS + sid

        @pl.when(j < NOCH)
        def _():
            pltpu.sync_copy(
                agg.at[pl.ds(j * OCH, OCH)],
                out_hbm.at[cid, pl.ds(j * OCH, OCH)],
            )

        return 0

    lax.fori_loop(0, (NOCH + NS - 1) // NS, out_chunk, 0)


_RB = 1000
_GRID = N // _RB


def _dis_from_degp(degp_ref):
    deg = degp_ref[0, :, 0] + degp_ref[1, :, 0] + 1.0
    return lax.rsqrt(deg)[:, None]


def _tc1_body(degp_ref, x_ref, w_ref, y_ref):
    dis = _dis_from_degp(degp_ref)
    y_ref[...] = dis * jnp.dot(
        x_ref[...], w_ref[...], preferred_element_type=jnp.float32
    )


def _tc2_body(degp_ref, p_ref, y1_ref, w_ref, b_ref, y2_ref):
    dis = _dis_from_degp(degp_ref)
    aggr = p_ref[0] + p_ref[1] + y1_ref[...]
    h1 = jnp.maximum(dis * aggr + b_ref[...], 0.0)
    y2_ref[...] = dis * jnp.dot(h1, w_ref[...], preferred_element_type=jnp.float32)


def _tc3_body(degp_ref, p_ref, y2_ref, b_ref, h2_ref):
    dis = _dis_from_degp(degp_ref)
    aggr = p_ref[0] + p_ref[1] + y2_ref[...]
    h2_ref[...] = jnp.maximum(dis * aggr + b_ref[...], 0.0)


_degp_spec = pl.BlockSpec((NC, _RB, D), lambda i: (0, i, 0))
_row_spec = pl.BlockSpec((_RB, D), lambda i: (i, 0))
_part_spec = pl.BlockSpec((NC, _RB, D), lambda i: (0, i, 0))
_w_spec = pl.BlockSpec((D, D), lambda i: (0, 0))
_b_spec = pl.BlockSpec((D,), lambda i: (0,))

_tc1 = pl.pallas_call(
    _tc1_body,
    grid=(_GRID,),
    in_specs=[_degp_spec, _row_spec, _w_spec],
    out_specs=_row_spec,
    out_shape=jax.ShapeDtypeStruct((N, D), jnp.float32),
)

_tc2 = pl.pallas_call(
    _tc2_body,
    grid=(_GRID,),
    in_specs=[_degp_spec, _part_spec, _row_spec, _w_spec, _b_spec],
    out_specs=_row_spec,
    out_shape=jax.ShapeDtypeStruct((N, D), jnp.float32),
)

_tc3 = pl.pallas_call(
    _tc3_body,
    grid=(_GRID,),
    in_specs=[_degp_spec, _part_spec, _row_spec, _b_spec],
    out_specs=_row_spec,
    out_shape=jax.ShapeDtypeStruct((N, D), jnp.float32),
)


def kernel(x, edge_index, W1, b1, W2, b2):
    src = edge_index[0].astype(jnp.int32)
    dst = edge_index[1].astype(jnp.int32)
    ept_real = E // NW
    pad = EPT - ept_real
    dum = jnp.broadcast_to(
        DUMMY + jnp.arange(pad, dtype=jnp.int32)[None, :], (NW, pad)
    )
    src_p = jnp.concatenate(
        [src.reshape(NW, ept_real), jnp.zeros((NW, pad), jnp.int32)], axis=1
    ).reshape(NW, NCH, CH)
    dst_p = jnp.concatenate([dst.reshape(NW, ept_real), dum], axis=1).reshape(
        NW, NCH, CH
    )

    degp = _sc_deg(dst_p)
    y1 = _tc1(degp, x, W1)
    p1 = _sc_agg(y1, src_p, dst_p)
    y2 = _tc2(degp, p1, y1, W2, b1)
    p2 = _sc_agg(y2, src_p, dst_p)
    h2 = _tc3(degp, p2, y2, b2)
    return h2


# speedup vs baseline: 1.6623x; 1.4149x over previous
import functools

import jax
import jax.numpy as jnp
from jax import lax
from jax.experimental import pallas as pl
from jax.experimental.pallas import tpu as pltpu
from jax.experimental.pallas import tpu_sc as plsc

N = 10000
E = 320000
D = 128

NC = 2
NS = 16
NW = NC * NS

CH = 128
NCH = 79
EPT = NCH * CH
E_PAD = NW * EPT
DUMMY = N
NDUM = 15

AGG_ROWS = 10240
ZROWS = 32
ROWS_PER_TILE_ZERO = AGG_ROWS // NS
OCH = 80
NOCH = N // OCH

_MESH = plsc.VectorSubcoreMesh(
    core_axis_name="c", subcore_axis_name="s", num_cores=NC, num_subcores=NS
)


def _zero_fill(ref, nrows, ncols):
    zv = jnp.zeros((16,), jnp.float32)

    def body(i, _):
        r = i // (ncols // 16)
        c = (i % (ncols // 16)) * 16
        ref[r, pl.ds(c, 16)] = zv
        return 0

    lax.fori_loop(0, nrows * (ncols // 16), body, 0)


@functools.partial(
    pl.kernel,
    out_type=jax.ShapeDtypeStruct((NC, N, D), jnp.float32),
    mesh=_MESH,
    scratch_types=[
        pltpu.VMEM((NCH, CH), jnp.int32),
        pltpu.VMEM((CH, D), jnp.float32),
        pltpu.VMEM((ZROWS, D), jnp.float32),
        pltpu.VMEM_SHARED((AGG_ROWS, D), jnp.float32),
    ],
)
def _sc_deg(dst_hbm, out_hbm, dstv, onesv, zbuf, dega):
    cid = lax.axis_index("c")
    sid = lax.axis_index("s")
    wid = sid * NC + cid

    _zero_fill(zbuf, ZROWS, D)
    _zero_fill(onesv, CH, D)
    e0 = jnp.where(lax.iota(jnp.int32, 16) == 0, 1.0, 0.0).astype(jnp.float32)

    def fill_ones(i, _):
        onesv[i, pl.ds(0, 16)] = e0
        return 0

    lax.fori_loop(0, CH, fill_ones, 0)

    zbase = sid * ROWS_PER_TILE_ZERO

    def zero_chunk(t, _):
        pltpu.sync_copy(zbuf, dega.at[pl.ds(zbase + t * ZROWS, ZROWS)])
        return 0

    lax.fori_loop(0, ROWS_PER_TILE_ZERO // ZROWS, zero_chunk, 0)

    pltpu.sync_copy(dst_hbm.at[wid], dstv)
    plsc.subcore_barrier()

    def edge_chunk(j, _):
        pltpu.sync_copy(onesv, dega.at[dstv.at[j]], add=True)
        return 0

    lax.fori_loop(0, NCH, edge_chunk, 0)
    plsc.subcore_barrier()

    def out_chunk(t, _):
        j = t * NS + sid

        @pl.when(j < NOCH)
        def _():
            pltpu.sync_copy(
                dega.at[pl.ds(j * OCH, OCH)],
                out_hbm.at[cid, pl.ds(j * OCH, OCH)],
            )

        return 0

    lax.fori_loop(0, (NOCH + NS - 1) // NS, out_chunk, 0)


@functools.partial(
    pl.kernel,
    out_type=jax.ShapeDtypeStruct((NC, N, D), jnp.float32),
    mesh=_MESH,
    scratch_types=[
        pltpu.VMEM((NCH, CH), jnp.int32),
        pltpu.VMEM((NCH, CH), jnp.int32),
        pltpu.VMEM((CH, D), jnp.float32),
        pltpu.VMEM((ZROWS, D), jnp.float32),
        pltpu.VMEM_SHARED((AGG_ROWS, D), jnp.float32),
        pltpu.SemaphoreType.DMA,
    ],
)
def _sc_agg(y_hbm, src_hbm, dst_hbm, out_hbm, srcv, dstv, rows, zbuf, agg, sem):
    cid = lax.axis_index("c")
    sid = lax.axis_index("s")
    wid = sid * NC + cid

    _zero_fill(zbuf, ZROWS, D)

    zbase = sid * ROWS_PER_TILE_ZERO

    def zero_chunk(t, _):
        pltpu.sync_copy(zbuf, agg.at[pl.ds(zbase + t * ZROWS, ZROWS)])
        return 0

    lax.fori_loop(0, ROWS_PER_TILE_ZERO // ZROWS, zero_chunk, 0)

    pltpu.sync_copy(src_hbm.at[wid], srcv)
    pltpu.sync_copy(dst_hbm.at[wid], dstv)
    plsc.subcore_barrier()

    def edge_chunk(j, _):
        pltpu.async_copy(y_hbm.at[srcv.at[j]], rows, sem).wait()
        pltpu.sync_copy(rows, agg.at[dstv.at[j]], add=True)
        return 0

    lax.fori_loop(0, NCH, edge_chunk, 0)
    plsc.subcore_barrier()

    def out_chunk(t, _):
        j = t * NS + sid

        @pl.when(j < NOCH)
        def _():
            pltpu.sync_copy(
                agg.at[pl.ds(j * OCH, OCH)],
                out_hbm.at[cid, pl.ds(j * OCH, OCH)],
            )

        return 0

    lax.fori_loop(0, (NOCH + NS - 1) // NS, out_chunk, 0)


_RB = 1000
_GRID = N // _RB


def _dis_from_degp(degp_ref):
    deg = degp_ref[0, :, 0] + degp_ref[1, :, 0] + 1.0
    return lax.rsqrt(deg)[:, None]


def _tc1_body(degp_ref, x_ref, w_ref, y_ref):
    dis = _dis_from_degp(degp_ref)
    y_ref[...] = dis * jnp.dot(
        x_ref[...], w_ref[...], preferred_element_type=jnp.float32
    )


def _tc2_body(degp_ref, p_ref, y1_ref, w_ref, b_ref, y2_ref):
    dis = _dis_from_degp(degp_ref)
    aggr = p_ref[0] + p_ref[1] + y1_ref[...]
    h1 = jnp.maximum(dis * aggr + b_ref[...], 0.0)
    y2_ref[...] = dis * jnp.dot(h1, w_ref[...], preferred_element_type=jnp.float32)


def _tc3_body(degp_ref, p_ref, y2_ref, b_ref, h2_ref):
    dis = _dis_from_degp(degp_ref)
    aggr = p_ref[0] + p_ref[1] + y2_ref[...]
    h2_ref[...] = jnp.maximum(dis * aggr + b_ref[...], 0.0)


_degp_spec = pl.BlockSpec((NC, _RB, D), lambda i: (0, i, 0))
_row_spec = pl.BlockSpec((_RB, D), lambda i: (i, 0))
_part_spec = pl.BlockSpec((NC, _RB, D), lambda i: (0, i, 0))
_w_spec = pl.BlockSpec((D, D), lambda i: (0, 0))
_b_spec = pl.BlockSpec((D,), lambda i: (0,))

_tc1 = pl.pallas_call(
    _tc1_body,
    grid=(_GRID,),
    in_specs=[_degp_spec, _row_spec, _w_spec],
    out_specs=_row_spec,
    out_shape=jax.ShapeDtypeStruct((N, D), jnp.float32),
)

_tc2 = pl.pallas_call(
    _tc2_body,
    grid=(_GRID,),
    in_specs=[_degp_spec, _part_spec, _row_spec, _w_spec, _b_spec],
    out_specs=_row_spec,
    out_shape=jax.ShapeDtypeStruct((N, D), jnp.float32),
)

_tc3 = pl.pallas_call(
    _tc3_body,
    grid=(_GRID,),
    in_specs=[_degp_spec, _part_spec, _row_spec, _b_spec],
    out_specs=_row_spec,
    out_shape=jax.ShapeDtypeStruct((N, D), jnp.float32),
)


def kernel(x, edge_index, W1, b1, W2, b2):
    src = edge_index[0].astype(jnp.int32)
    dst = edge_index[1].astype(jnp.int32)
    ept_real = E // NW
    pad = EPT - ept_real
    sid = jnp.arange(NW, dtype=jnp.int32)[:, None] // NC
    dum = DUMMY + sid * NDUM + jnp.arange(pad, dtype=jnp.int32)[None, :] % NDUM
    src_p = jnp.concatenate(
        [src.reshape(NW, ept_real), jnp.zeros((NW, pad), jnp.int32)], axis=1
    ).reshape(NW, NCH, CH)
    dst_p = jnp.concatenate([dst.reshape(NW, ept_real), dum], axis=1).reshape(
        NW, NCH, CH
    )

    degp = _sc_deg(dst_p)
    y1 = _tc1(degp, x, W1)
    p1 = _sc_agg(y1, src_p, dst_p)
    y2 = _tc2(degp, p1, y1, W2, b1)
    p2 = _sc_agg(y2, src_p, dst_p)
    h2 = _tc3(degp, p2, y2, b2)
    return h2


# speedup vs baseline: 1.8742x; 1.1274x over previous
import functools

import jax
import jax.numpy as jnp
from jax import lax
from jax.experimental import pallas as pl
from jax.experimental.pallas import tpu as pltpu
from jax.experimental.pallas import tpu_sc as plsc

N = 10000
E = 320000
D = 128

NC = 2
NS = 16
NW = NC * NS

CH = 128
NCH = 80
NCHL = 79
NCHB = 16
NBLK = NCH // NCHB
EPT = NCH * CH
E_PAD = NW * EPT
DUMMY = N
NDUM = 15

AGG_ROWS = 10240
ZROWS = 16
ROWS_PER_TILE_ZERO = AGG_ROWS // NS
OCH = 80
NOCH = N // OCH

_MESH = plsc.VectorSubcoreMesh(
    core_axis_name="c", subcore_axis_name="s", num_cores=NC, num_subcores=NS
)


def _zero_fill(ref, nrows, ncols):
    zv = jnp.zeros((16,), jnp.float32)

    def body(i, _):
        r = i // (ncols // 16)
        c = (i % (ncols // 16)) * 16
        ref[r, pl.ds(c, 16)] = zv
        return 0

    lax.fori_loop(0, nrows * (ncols // 16), body, 0)


@functools.partial(
    pl.kernel,
    out_type=jax.ShapeDtypeStruct((NC, N, D), jnp.float32),
    mesh=_MESH,
    scratch_types=[
        pltpu.VMEM((NCH, CH), jnp.int32),
        pltpu.VMEM((CH, D), jnp.float32),
        pltpu.VMEM((ZROWS, D), jnp.float32),
        pltpu.VMEM_SHARED((AGG_ROWS, D), jnp.float32),
    ],
)
def _sc_deg(dst_hbm, out_hbm, dstv, onesv, zbuf, dega):
    cid = lax.axis_index("c")
    sid = lax.axis_index("s")
    wid = sid * NC + cid

    _zero_fill(zbuf, ZROWS, D)
    _zero_fill(onesv, CH, D)
    e0 = jnp.where(lax.iota(jnp.int32, 16) == 0, 1.0, 0.0).astype(jnp.float32)

    def fill_ones(i, _):
        onesv[i, pl.ds(0, 16)] = e0
        return 0

    lax.fori_loop(0, CH, fill_ones, 0)

    zbase = sid * ROWS_PER_TILE_ZERO

    def zero_chunk(t, _):
        pltpu.sync_copy(zbuf, dega.at[pl.ds(zbase + t * ZROWS, ZROWS)])
        return 0

    lax.fori_loop(0, ROWS_PER_TILE_ZERO // ZROWS, zero_chunk, 0)

    pltpu.sync_copy(dst_hbm.at[wid], dstv)
    plsc.subcore_barrier()

    def edge_chunk(j, _):
        pltpu.sync_copy(onesv, dega.at[dstv.at[j]], add=True)
        return 0

    lax.fori_loop(0, NCHL, edge_chunk, 0)
    plsc.subcore_barrier()

    def out_chunk(t, _):
        j = t * NS + sid

        @pl.when(j < NOCH)
        def _():
            pltpu.sync_copy(
                dega.at[pl.ds(j * OCH, OCH)],
                out_hbm.at[cid, pl.ds(j * OCH, OCH)],
            )

        return 0

    lax.fori_loop(0, (NOCH + NS - 1) // NS, out_chunk, 0)


@functools.partial(
    pl.kernel,
    out_type=jax.ShapeDtypeStruct((NC, N, D), jnp.float32),
    mesh=_MESH,
    scratch_types=[
        pltpu.VMEM((2, NCHB, CH), jnp.int32),
        pltpu.VMEM((2, NCHB, CH), jnp.int32),
        pltpu.VMEM((2, CH, D), jnp.float32),
        pltpu.VMEM((ZROWS, D), jnp.float32),
        pltpu.VMEM_SHARED((AGG_ROWS, D), jnp.float32),
        pltpu.SemaphoreType.DMA,
    ],
)
def _sc_agg(y_hbm, src_hbm, dst_hbm, out_hbm, srcv, dstv, rows, zbuf, agg, sem):
    cid = lax.axis_index("c")
    sid = lax.axis_index("s")
    wid = sid * NC + cid

    _zero_fill(zbuf, ZROWS, D)

    zbase = sid * ROWS_PER_TILE_ZERO

    def zero_chunk(t, _):
        pltpu.sync_copy(zbuf, agg.at[pl.ds(zbase + t * ZROWS, ZROWS)])
        return 0

    lax.fori_loop(0, ROWS_PER_TILE_ZERO // ZROWS, zero_chunk, 0)

    pltpu.sync_copy(src_hbm.at[wid, pl.ds(0, NCHB)], srcv.at[0])
    pltpu.sync_copy(dst_hbm.at[wid, pl.ds(0, NCHB)], dstv.at[0])
    plsc.subcore_barrier()

    pltpu.async_copy(y_hbm.at[srcv.at[0, 0]], rows.at[0], sem)

    def edge_chunk(j, _):
        blk = j // NCHB
        pos = lax.rem(j, NCHB)
        ib = lax.rem(blk, 2)
        buf = lax.rem(j, 2)

        @pl.when(jnp.logical_and(pos == 0, blk < NBLK - 1))
        def _():
            nb = blk + 1
            pltpu.sync_copy(src_hbm.at[wid, pl.ds(nb * NCHB, NCHB)], srcv.at[1 - ib])
            pltpu.sync_copy(dst_hbm.at[wid, pl.ds(nb * NCHB, NCHB)], dstv.at[1 - ib])

        pltpu.make_async_copy(y_hbm.at[srcv.at[ib, pos]], rows.at[buf], sem).wait()

        @pl.when(j < NCHL - 1)
        def _():
            jn = j + 1
            pltpu.async_copy(
                y_hbm.at[srcv.at[lax.rem(jn // NCHB, 2), lax.rem(jn, NCHB)]],
                rows.at[1 - buf],
                sem,
            )

        pltpu.sync_copy(rows.at[buf], agg.at[dstv.at[ib, pos]], add=True)
        return 0

    lax.fori_loop(0, NCHL, edge_chunk, 0)
    plsc.subcore_barrier()

    def out_chunk(t, _):
        j = t * NS + sid

        @pl.when(j < NOCH)
        def _():
            pltpu.sync_copy(
                agg.at[pl.ds(j * OCH, OCH)],
                out_hbm.at[cid, pl.ds(j * OCH, OCH)],
            )

        return 0

    lax.fori_loop(0, (NOCH + NS - 1) // NS, out_chunk, 0)


_RB = 1000
_GRID = N // _RB


def _dis_from_degp(degp_ref):
    deg = degp_ref[0, :, 0] + degp_ref[1, :, 0] + 1.0
    return lax.rsqrt(deg)[:, None]


def _tc1_body(degp_ref, x_ref, w_ref, y_ref):
    dis = _dis_from_degp(degp_ref)
    y_ref[...] = dis * jnp.dot(
        x_ref[...], w_ref[...], preferred_element_type=jnp.float32
    )


def _tc2_body(degp_ref, p_ref, y1_ref, w_ref, b_ref, y2_ref):
    dis = _dis_from_degp(degp_ref)
    aggr = p_ref[0] + p_ref[1] + y1_ref[...]
    h1 = jnp.maximum(dis * aggr + b_ref[...], 0.0)
    y2_ref[...] = dis * jnp.dot(h1, w_ref[...], preferred_element_type=jnp.float32)


def _tc3_body(degp_ref, p_ref, y2_ref, b_ref, h2_ref):
    dis = _dis_from_degp(degp_ref)
    aggr = p_ref[0] + p_ref[1] + y2_ref[...]
    h2_ref[...] = jnp.maximum(dis * aggr + b_ref[...], 0.0)


_degp_spec = pl.BlockSpec((NC, _RB, D), lambda i: (0, i, 0))
_row_spec = pl.BlockSpec((_RB, D), lambda i: (i, 0))
_part_spec = pl.BlockSpec((NC, _RB, D), lambda i: (0, i, 0))
_w_spec = pl.BlockSpec((D, D), lambda i: (0, 0))
_b_spec = pl.BlockSpec((D,), lambda i: (0,))

_tc1 = pl.pallas_call(
    _tc1_body,
    grid=(_GRID,),
    in_specs=[_degp_spec, _row_spec, _w_spec],
    out_specs=_row_spec,
    out_shape=jax.ShapeDtypeStruct((N, D), jnp.float32),
)

_tc2 = pl.pallas_call(
    _tc2_body,
    grid=(_GRID,),
    in_specs=[_degp_spec, _part_spec, _row_spec, _w_spec, _b_spec],
    out_specs=_row_spec,
    out_shape=jax.ShapeDtypeStruct((N, D), jnp.float32),
)

_tc3 = pl.pallas_call(
    _tc3_body,
    grid=(_GRID,),
    in_specs=[_degp_spec, _part_spec, _row_spec, _b_spec],
    out_specs=_row_spec,
    out_shape=jax.ShapeDtypeStruct((N, D), jnp.float32),
)


def kernel(x, edge_index, W1, b1, W2, b2):
    src = edge_index[0].astype(jnp.int32)
    dst = edge_index[1].astype(jnp.int32)
    ept_real = E // NW
    pad = EPT - ept_real
    sid = jnp.arange(NW, dtype=jnp.int32)[:, None] // NC
    dum = DUMMY + sid * NDUM + jnp.arange(pad, dtype=jnp.int32)[None, :] % NDUM
    src_p = jnp.concatenate(
        [src.reshape(NW, ept_real), jnp.zeros((NW, pad), jnp.int32)], axis=1
    ).reshape(NW, NCH, CH)
    dst_p = jnp.concatenate([dst.reshape(NW, ept_real), dum], axis=1).reshape(
        NW, NCH, CH
    )

    degp = _sc_deg(dst_p)
    y1 = _tc1(degp, x, W1)
    p1 = _sc_agg(y1, src_p, dst_p)
    y2 = _tc2(degp, p1, y1, W2, b1)
    p2 = _sc_agg(y2, src_p, dst_p)
    h2 = _tc3(degp, p2, y2, b2)
    return h2


# speedup vs baseline: 1.8860x; 1.0063x over previous
import functools

import jax
import jax.numpy as jnp
from jax import lax
from jax.experimental import pallas as pl
from jax.experimental.pallas import tpu as pltpu
from jax.experimental.pallas import tpu_sc as plsc

N = 10000
E = 320000
D = 128

NC = 2
NS = 16
NW = NC * NS

CH = 128
NCH = 80
NCHL = 79
NCHB = 16
NBLK = NCH // NCHB
EPT = NCH * CH
E_PAD = NW * EPT
DUMMY = N
NDUM = 15

AGG_ROWS = 10240
ZROWS = 16
ROWS_PER_TILE_ZERO = AGG_ROWS // NS
OCH = 80
NOCH = N // OCH

_MESH = plsc.VectorSubcoreMesh(
    core_axis_name="c", subcore_axis_name="s", num_cores=NC, num_subcores=NS
)


def _zero_fill(ref, nrows, ncols):
    zv = jnp.zeros((16,), jnp.float32)

    def body(i, _):
        r = i // (ncols // 16)
        c = (i % (ncols // 16)) * 16
        ref[r, pl.ds(c, 16)] = zv
        return 0

    lax.fori_loop(0, nrows * (ncols // 16), body, 0)


@functools.partial(
    pl.kernel,
    out_type=jax.ShapeDtypeStruct((NC, N, D), jnp.float32),
    mesh=_MESH,
    scratch_types=[
        pltpu.VMEM((NCH, CH), jnp.int32),
        pltpu.VMEM((CH, D), jnp.float32),
        pltpu.VMEM((ZROWS, D), jnp.float32),
        pltpu.VMEM_SHARED((AGG_ROWS, D), jnp.float32),
    ],
)
def _sc_deg(dst_hbm, out_hbm, dstv, onesv, zbuf, dega):
    cid = lax.axis_index("c")
    sid = lax.axis_index("s")
    wid = sid * NC + cid

    _zero_fill(zbuf, ZROWS, D)
    _zero_fill(onesv, CH, D)
    e0 = jnp.where(lax.iota(jnp.int32, 16) == 0, 1.0, 0.0).astype(jnp.float32)

    def fill_ones(i, _):
        onesv[i, pl.ds(0, 16)] = e0
        return 0

    lax.fori_loop(0, CH, fill_ones, 0)

    zbase = sid * ROWS_PER_TILE_ZERO

    def zero_chunk(t, _):
        pltpu.sync_copy(zbuf, dega.at[pl.ds(zbase + t * ZROWS, ZROWS)])
        return 0

    lax.fori_loop(0, ROWS_PER_TILE_ZERO // ZROWS, zero_chunk, 0)

    pltpu.sync_copy(dst_hbm.at[wid], dstv)
    plsc.subcore_barrier()

    def edge_chunk(j, _):
        pltpu.sync_copy(onesv, dega.at[dstv.at[j]], add=True)
        return 0

    lax.fori_loop(0, NCHL, edge_chunk, 0)
    plsc.subcore_barrier()

    def out_chunk(t, _):
        j = t * NS + sid

        @pl.when(j < NOCH)
        def _():
            pltpu.sync_copy(
                dega.at[pl.ds(j * OCH, OCH)],
                out_hbm.at[cid, pl.ds(j * OCH, OCH)],
            )

        return 0

    lax.fori_loop(0, (NOCH + NS - 1) // NS, out_chunk, 0)


@functools.partial(
    pl.kernel,
    out_type=jax.ShapeDtypeStruct((NC, N, D), jnp.float32),
    mesh=_MESH,
    scratch_types=[
        pltpu.VMEM((2, NCHB, CH), jnp.int32),
        pltpu.VMEM((2, NCHB, CH), jnp.int32),
        pltpu.VMEM((2, CH, D), jnp.float32),
        pltpu.VMEM((ZROWS, D), jnp.float32),
        pltpu.VMEM_SHARED((AGG_ROWS, D), jnp.float32),
        pltpu.SemaphoreType.DMA,
        pltpu.SemaphoreType.DMA,
    ],
)
def _sc_agg(y_hbm, src_hbm, dst_hbm, out_hbm, srcv, dstv, rows, zbuf, agg, sem, sems):
    cid = lax.axis_index("c")
    sid = lax.axis_index("s")
    wid = sid * NC + cid

    _zero_fill(zbuf, ZROWS, D)

    zbase = sid * ROWS_PER_TILE_ZERO

    def zero_chunk(t, _):
        pltpu.sync_copy(zbuf, agg.at[pl.ds(zbase + t * ZROWS, ZROWS)])
        return 0

    lax.fori_loop(0, ROWS_PER_TILE_ZERO // ZROWS, zero_chunk, 0)

    pltpu.sync_copy(src_hbm.at[wid, pl.ds(0, NCHB)], srcv.at[0])
    pltpu.sync_copy(dst_hbm.at[wid, pl.ds(0, NCHB)], dstv.at[0])
    plsc.subcore_barrier()

    pltpu.async_copy(y_hbm.at[srcv.at[0, 0]], rows.at[0], sem)

    def edge_chunk(j, _):
        blk = j // NCHB
        pos = lax.rem(j, NCHB)
        ib = lax.rem(blk, 2)
        buf = lax.rem(j, 2)

        @pl.when(jnp.logical_and(pos == 0, blk < NBLK - 1))
        def _():
            nb = blk + 1
            pltpu.sync_copy(src_hbm.at[wid, pl.ds(nb * NCHB, NCHB)], srcv.at[1 - ib])
            pltpu.sync_copy(dst_hbm.at[wid, pl.ds(nb * NCHB, NCHB)], dstv.at[1 - ib])

        pltpu.make_async_copy(y_hbm.at[srcv.at[ib, pos]], rows.at[buf], sem).wait()
        pltpu.async_copy(rows.at[buf], agg.at[dstv.at[ib, pos]], sems, add=True)

        @pl.when(j >= 1)
        def _():
            jp = j - 1
            pltpu.make_async_copy(
                rows.at[1 - buf],
                agg.at[dstv.at[lax.rem(jp // NCHB, 2), lax.rem(jp, NCHB)]],
                sems,
            ).wait()

        @pl.when(j < NCHL - 1)
        def _():
            jn = j + 1
            pltpu.async_copy(
                y_hbm.at[srcv.at[lax.rem(jn // NCHB, 2), lax.rem(jn, NCHB)]],
                rows.at[1 - buf],
                sem,
            )

        return 0

    lax.fori_loop(0, NCHL, edge_chunk, 0)
    jl = NCHL - 1
    pltpu.make_async_copy(
        rows.at[lax.rem(jl, 2)],
        agg.at[dstv.at[lax.rem(jl // NCHB, 2), lax.rem(jl, NCHB)]],
        sems,
    ).wait()
    plsc.subcore_barrier()

    def out_chunk(t, _):
        j = t * NS + sid

        @pl.when(j < NOCH)
        def _():
            pltpu.sync_copy(
                agg.at[pl.ds(j * OCH, OCH)],
                out_hbm.at[cid, pl.ds(j * OCH, OCH)],
            )

        return 0

    lax.fori_loop(0, (NOCH + NS - 1) // NS, out_chunk, 0)


_RB = 1000
_GRID = N // _RB


def _dis_from_degp(degp_ref):
    deg = degp_ref[0, :, 0] + degp_ref[1, :, 0] + 1.0
    return lax.rsqrt(deg)[:, None]


def _tc1_body(degp_ref, x_ref, w_ref, y_ref):
    dis = _dis_from_degp(degp_ref)
    y_ref[...] = dis * jnp.dot(
        x_ref[...], w_ref[...], preferred_element_type=jnp.float32
    )


def _tc2_body(degp_ref, p_ref, y1_ref, w_ref, b_ref, y2_ref):
    dis = _dis_from_degp(degp_ref)
    aggr = p_ref[0] + p_ref[1] + y1_ref[...]
    h1 = jnp.maximum(dis * aggr + b_ref[...], 0.0)
    y2_ref[...] = dis * jnp.dot(h1, w_ref[...], preferred_element_type=jnp.float32)


def _tc3_body(degp_ref, p_ref, y2_ref, b_ref, h2_ref):
    dis = _dis_from_degp(degp_ref)
    aggr = p_ref[0] + p_ref[1] + y2_ref[...]
    h2_ref[...] = jnp.maximum(dis * aggr + b_ref[...], 0.0)


_degp_spec = pl.BlockSpec((NC, _RB, D), lambda i: (0, i, 0))
_row_spec = pl.BlockSpec((_RB, D), lambda i: (i, 0))
_part_spec = pl.BlockSpec((NC, _RB, D), lambda i: (0, i, 0))
_w_spec = pl.BlockSpec((D, D), lambda i: (0, 0))
_b_spec = pl.BlockSpec((D,), lambda i: (0,))

_tc1 = pl.pallas_call(
    _tc1_body,
    grid=(_GRID,),
    in_specs=[_degp_spec, _row_spec, _w_spec],
    out_specs=_row_spec,
    out_shape=jax.ShapeDtypeStruct((N, D), jnp.float32),
)

_tc2 = pl.pallas_call(
    _tc2_body,
    grid=(_GRID,),
    in_specs=[_degp_spec, _part_spec, _row_spec, _w_spec, _b_spec],
    out_specs=_row_spec,
    out_shape=jax.ShapeDtypeStruct((N, D), jnp.float32),
)

_tc3 = pl.pallas_call(
    _tc3_body,
    grid=(_GRID,),
    in_specs=[_degp_spec, _part_spec, _row_spec, _b_spec],
    out_specs=_row_spec,
    out_shape=jax.ShapeDtypeStruct((N, D), jnp.float32),
)


def kernel(x, edge_index, W1, b1, W2, b2):
    src = edge_index[0].astype(jnp.int32)
    dst = edge_index[1].astype(jnp.int32)
    ept_real = E // NW
    pad = EPT - ept_real
    sid = jnp.arange(NW, dtype=jnp.int32)[:, None] // NC
    dum = DUMMY + sid * NDUM + jnp.arange(pad, dtype=jnp.int32)[None, :] % NDUM
    src_p = jnp.concatenate(
        [src.reshape(NW, ept_real), jnp.zeros((NW, pad), jnp.int32)], axis=1
    ).reshape(NW, NCH, CH)
    dst_p = jnp.concatenate([dst.reshape(NW, ept_real), dum], axis=1).reshape(
        NW, NCH, CH
    )

    degp = _sc_deg(dst_p)
    y1 = _tc1(degp, x, W1)
    p1 = _sc_agg(y1, src_p, dst_p)
    y2 = _tc2(degp, p1, y1, W2, b1)
    p2 = _sc_agg(y2, src_p, dst_p)
    h2 = _tc3(degp, p2, y2, b2)
    return h2


# speedup vs baseline: 1.8941x; 1.0043x over previous
import functools

import jax
import jax.numpy as jnp
from jax import lax
from jax.experimental import pallas as pl
from jax.experimental.pallas import tpu as pltpu
from jax.experimental.pallas import tpu_sc as plsc

N = 10000
E = 320000
D = 128

NC = 2
NS = 16
NW = NC * NS

CH = 128
NCH = 80
NCHL = 79
NCHB = 16
NBLK = NCH // NCHB
EPT = NCH * CH
E_PAD = NW * EPT
DUMMY = N
NDUM = 15

AGG_ROWS = 10240
ZROWS = 16
ROWS_PER_TILE_ZERO = AGG_ROWS // NS
OCH = 80
NOCH = N // OCH

_MESH = plsc.VectorSubcoreMesh(
    core_axis_name="c", subcore_axis_name="s", num_cores=NC, num_subcores=NS
)


def _zero_shared(zbuf, shared, zbase, sem):
    nz = ROWS_PER_TILE_ZERO // ZROWS

    def fire(t, _):
        pltpu.async_copy(zbuf, shared.at[pl.ds(zbase + t * ZROWS, ZROWS)], sem)
        return 0

    def drain(t, _):
        pltpu.make_async_copy(
            zbuf, shared.at[pl.ds(zbase + t * ZROWS, ZROWS)], sem
        ).wait()
        return 0

    lax.fori_loop(0, nz, fire, 0)
    lax.fori_loop(0, nz, drain, 0)


def _copy_out(shared, out_hbm, cid, sid, sem):
    nt = (NOCH + NS - 1) // NS

    def fire(t, _):
        j = t * NS + sid

        @pl.when(j < NOCH)
        def _():
            pltpu.async_copy(
                shared.at[pl.ds(j * OCH, OCH)],
                out_hbm.at[cid, pl.ds(j * OCH, OCH)],
                sem,
            )

        return 0

    def drain(t, _):
        j = t * NS + sid

        @pl.when(j < NOCH)
        def _():
            pltpu.make_async_copy(
                shared.at[pl.ds(j * OCH, OCH)],
                out_hbm.at[cid, pl.ds(j * OCH, OCH)],
                sem,
            ).wait()

        return 0

    lax.fori_loop(0, nt, fire, 0)
    lax.fori_loop(0, nt, drain, 0)


def _zero_fill(ref, nrows, ncols):
    zv = jnp.zeros((16,), jnp.float32)

    def body(i, _):
        r = i // (ncols // 16)
        c = (i % (ncols // 16)) * 16
        ref[r, pl.ds(c, 16)] = zv
        return 0

    lax.fori_loop(0, nrows * (ncols // 16), body, 0)


@functools.partial(
    pl.kernel,
    out_type=jax.ShapeDtypeStruct((NC, N, D), jnp.float32),
    mesh=_MESH,
    scratch_types=[
        pltpu.VMEM((NCH, CH), jnp.int32),
        pltpu.VMEM((CH, D), jnp.float32),
        pltpu.VMEM((ZROWS, D), jnp.float32),
        pltpu.VMEM_SHARED((AGG_ROWS, D), jnp.float32),
        pltpu.SemaphoreType.DMA,
    ],
)
def _sc_deg(dst_hbm, out_hbm, dstv, onesv, zbuf, dega, sem):
    cid = lax.axis_index("c")
    sid = lax.axis_index("s")
    wid = sid * NC + cid

    _zero_fill(zbuf, ZROWS, D)
    _zero_fill(onesv, CH, D)
    e0 = jnp.where(lax.iota(jnp.int32, 16) == 0, 1.0, 0.0).astype(jnp.float32)

    def fill_ones(i, _):
        onesv[i, pl.ds(0, 16)] = e0
        return 0

    lax.fori_loop(0, CH, fill_ones, 0)

    zbase = sid * ROWS_PER_TILE_ZERO

    _zero_shared(zbuf, dega, zbase, sem)

    pltpu.sync_copy(dst_hbm.at[wid], dstv)
    plsc.subcore_barrier()

    def edge_chunk(j, _):
        pltpu.sync_copy(onesv, dega.at[dstv.at[j]], add=True)
        return 0

    lax.fori_loop(0, NCHL, edge_chunk, 0)
    plsc.subcore_barrier()

    _copy_out(dega, out_hbm, cid, sid, sem)


@functools.partial(
    pl.kernel,
    out_type=jax.ShapeDtypeStruct((NC, N, D), jnp.float32),
    mesh=_MESH,
    scratch_types=[
        pltpu.VMEM((2, NCHB, CH), jnp.int32),
        pltpu.VMEM((2, NCHB, CH), jnp.int32),
        pltpu.VMEM((2, CH, D), jnp.float32),
        pltpu.VMEM((ZROWS, D), jnp.float32),
        pltpu.VMEM_SHARED((AGG_ROWS, D), jnp.float32),
        pltpu.SemaphoreType.DMA,
        pltpu.SemaphoreType.DMA,
    ],
)
def _sc_agg(y_hbm, src_hbm, dst_hbm, out_hbm, srcv, dstv, rows, zbuf, agg, sem, sems):
    cid = lax.axis_index("c")
    sid = lax.axis_index("s")
    wid = sid * NC + cid

    _zero_fill(zbuf, ZROWS, D)

    zbase = sid * ROWS_PER_TILE_ZERO

    _zero_shared(zbuf, agg, zbase, sem)

    pltpu.sync_copy(src_hbm.at[wid, pl.ds(0, NCHB)], srcv.at[0])
    pltpu.sync_copy(dst_hbm.at[wid, pl.ds(0, NCHB)], dstv.at[0])
    plsc.subcore_barrier()

    pltpu.async_copy(y_hbm.at[srcv.at[0, 0]], rows.at[0], sem)

    def edge_chunk(j, _):
        blk = j // NCHB
        pos = lax.rem(j, NCHB)
        ib = lax.rem(blk, 2)
        buf = lax.rem(j, 2)

        @pl.when(jnp.logical_and(pos == 0, blk < NBLK - 1))
        def _():
            nb = blk + 1
            pltpu.sync_copy(src_hbm.at[wid, pl.ds(nb * NCHB, NCHB)], srcv.at[1 - ib])
            pltpu.sync_copy(dst_hbm.at[wid, pl.ds(nb * NCHB, NCHB)], dstv.at[1 - ib])

        pltpu.make_async_copy(y_hbm.at[srcv.at[ib, pos]], rows.at[buf], sem).wait()
        pltpu.async_copy(rows.at[buf], agg.at[dstv.at[ib, pos]], sems, add=True)

        @pl.when(j >= 1)
        def _():
            jp = j - 1
            pltpu.make_async_copy(
                rows.at[1 - buf],
                agg.at[dstv.at[lax.rem(jp // NCHB, 2), lax.rem(jp, NCHB)]],
                sems,
            ).wait()

        @pl.when(j < NCHL - 1)
        def _():
            jn = j + 1
            pltpu.async_copy(
                y_hbm.at[srcv.at[lax.rem(jn // NCHB, 2), lax.rem(jn, NCHB)]],
                rows.at[1 - buf],
                sem,
            )

        return 0

    lax.fori_loop(0, NCHL, edge_chunk, 0)
    jl = NCHL - 1
    pltpu.make_async_copy(
        rows.at[lax.rem(jl, 2)],
        agg.at[dstv.at[lax.rem(jl // NCHB, 2), lax.rem(jl, NCHB)]],
        sems,
    ).wait()
    plsc.subcore_barrier()

    _copy_out(agg, out_hbm, cid, sid, sem)


_RB = 1000
_GRID = N // _RB


def _dis_from_degp(degp_ref):
    deg = degp_ref[0, :, 0] + degp_ref[1, :, 0] + 1.0
    return lax.rsqrt(deg)[:, None]


def _tc1_body(degp_ref, x_ref, w_ref, y_ref):
    dis = _dis_from_degp(degp_ref)
    y_ref[...] = dis * jnp.dot(
        x_ref[...], w_ref[...], preferred_element_type=jnp.float32
    )


def _tc2_body(degp_ref, p_ref, y1_ref, w_ref, b_ref, y2_ref):
    dis = _dis_from_degp(degp_ref)
    aggr = p_ref[0] + p_ref[1] + y1_ref[...]
    h1 = jnp.maximum(dis * aggr + b_ref[...], 0.0)
    y2_ref[...] = dis * jnp.dot(h1, w_ref[...], preferred_element_type=jnp.float32)


def _tc3_body(degp_ref, p_ref, y2_ref, b_ref, h2_ref):
    dis = _dis_from_degp(degp_ref)
    aggr = p_ref[0] + p_ref[1] + y2_ref[...]
    h2_ref[...] = jnp.maximum(dis * aggr + b_ref[...], 0.0)


_degp_spec = pl.BlockSpec((NC, _RB, D), lambda i: (0, i, 0))
_row_spec = pl.BlockSpec((_RB, D), lambda i: (i, 0))
_part_spec = pl.BlockSpec((NC, _RB, D), lambda i: (0, i, 0))
_w_spec = pl.BlockSpec((D, D), lambda i: (0, 0))
_b_spec = pl.BlockSpec((D,), lambda i: (0,))

_tc1 = pl.pallas_call(
    _tc1_body,
    grid=(_GRID,),
    in_specs=[_degp_spec, _row_spec, _w_spec],
    out_specs=_row_spec,
    out_shape=jax.ShapeDtypeStruct((N, D), jnp.float32),
)

_tc2 = pl.pallas_call(
    _tc2_body,
    grid=(_GRID,),
    in_specs=[_degp_spec, _part_spec, _row_spec, _w_spec, _b_spec],
    out_specs=_row_spec,
    out_shape=jax.ShapeDtypeStruct((N, D), jnp.float32),
)

_tc3 = pl.pallas_call(
    _tc3_body,
    grid=(_GRID,),
    in_specs=[_degp_spec, _part_spec, _row_spec, _b_spec],
    out_specs=_row_spec,
    out_shape=jax.ShapeDtypeStruct((N, D), jnp.float32),
)


def kernel(x, edge_index, W1, b1, W2, b2):
    src = edge_index[0].astype(jnp.int32)
    dst = edge_index[1].astype(jnp.int32)
    ept_real = E // NW
    pad = EPT - ept_real
    sid = jnp.arange(NW, dtype=jnp.int32)[:, None] // NC
    dum = DUMMY + sid * NDUM + jnp.arange(pad, dtype=jnp.int32)[None, :] % NDUM
    src_p = jnp.concatenate(
        [src.reshape(NW, ept_real), jnp.zeros((NW, pad), jnp.int32)], axis=1
    ).reshape(NW, NCH, CH)
    dst_p = jnp.concatenate([dst.reshape(NW, ept_real), dum], axis=1).reshape(
        NW, NCH, CH
    )

    degp = _sc_deg(dst_p)
    y1 = _tc1(degp, x, W1)
    p1 = _sc_agg(y1, src_p, dst_p)
    y2 = _tc2(degp, p1, y1, W2, b1)
    p2 = _sc_agg(y2, src_p, dst_p)
    h2 = _tc3(degp, p2, y2, b2)
    return h2
